# baseline (speedup 1.0000x reference)
import jax
import jax.numpy as jnp
from jax.experimental import pallas as pl
from jax.experimental.pallas import tpu as pltpu

TOKENS = 32768
HIDDEN = 4096
N_EXPERTS = 64
TOP_K = 8
TILE = 1024
HALF = HIDDEN // 2


def _gate_kernel(xa_ref, xb_ref, w_ref, probs_ref, vals_ref, idx_ref):
    w = w_ref[...]
    dn = (((1,), (1,)), ((), ()))
    logits = jax.lax.dot_general(
        xa_ref[...], w[:, :HALF], dn, preferred_element_type=jnp.float32
    ) + jax.lax.dot_general(
        xb_ref[...], w[:, HALF:], dn, preferred_element_type=jnp.float32
    )
    m = jnp.max(logits, axis=-1, keepdims=True)
    e = jnp.exp(logits - m)
    s = jnp.sum(e, axis=-1, keepdims=True)
    probs = e / s
    probs_ref[...] = probs

    work = probs
    iota = jax.lax.broadcasted_iota(jnp.int32, probs.shape, 1).astype(
        jnp.float32
    )
    vals = []
    idxs = []
    for _ in range(TOP_K):
        v = jnp.max(work, axis=-1, keepdims=True)
        i = jnp.min(
            jnp.where(work == v, iota, float(N_EXPERTS)),
            axis=-1,
            keepdims=True,
        )
        vals.append(v)
        idxs.append(i)
        work = jnp.where(iota == i, -jnp.inf, work)
    top_vals = jnp.concatenate(vals, axis=-1)
    top_idx = jnp.concatenate(idxs, axis=-1)
    top_vals = top_vals / jnp.sum(top_vals, axis=-1, keepdims=True)
    vals_ref[...] = top_vals
    idx_ref[...] = top_idx.astype(jnp.int32)


@jax.jit
def kernel(x, W):
    n_tiles = TOKENS // TILE
    probs, top_vals, top_idx = pl.pallas_call(
        _gate_kernel,
        grid=(n_tiles,),
        in_specs=[
            pl.BlockSpec((TILE, HALF), lambda i: (i, 0)),
            pl.BlockSpec((TILE, HALF), lambda i: (i, 1)),
            pl.BlockSpec((N_EXPERTS, HIDDEN), lambda i: (0, 0)),
        ],
        out_specs=[
            pl.BlockSpec((TILE, N_EXPERTS), lambda i: (i, 0)),
            pl.BlockSpec((TILE, TOP_K), lambda i: (i, 0)),
            pl.BlockSpec((TILE, TOP_K), lambda i: (i, 0)),
        ],
        out_shape=[
            jax.ShapeDtypeStruct((TOKENS, N_EXPERTS), jnp.float32),
            jax.ShapeDtypeStruct((TOKENS, TOP_K), jnp.float32),
            jax.ShapeDtypeStruct((TOKENS, TOP_K), jnp.int32),
        ],
        compiler_params=pltpu.CompilerParams(
            dimension_semantics=("parallel",),
        ),
    )(x, x, W)
    return (probs, top_vals, top_idx)
